# baseline (speedup 1.0000x reference)
import functools

import jax
import jax.numpy as jnp
from jax import lax
from jax.experimental import pallas as pl
from jax.experimental.pallas import tpu as pltpu
from jax.experimental.pallas import tpu_sc as plsc

N = 10000
D = 256
E = 160000
H = D // 2
NC = 2
NS = 16
L = 16
CH = 112
NCHUNK = 90
EPT = CH * NCHUNK
EPAD = NS * EPT
PADR = 16
NZT = 10
RPT = N // NZT


@functools.cache
def _make_edge_scatter():
    return functools.partial(
        pl.kernel,
        out_type=jax.ShapeDtypeStruct((NC, N, H), jnp.float32),
        mesh=plsc.VectorSubcoreMesh(
            core_axis_name="c", subcore_axis_name="s",
            num_cores=NC, num_subcores=NS,
        ),
        scratch_types=[
            pltpu.VMEM_SHARED((N + PADR, H), jnp.float32),
            pltpu.VMEM((CH,), jnp.int32),
            pltpu.VMEM((CH,), jnp.int32),
            pltpu.VMEM((CH,), jnp.int32),
            pltpu.VMEM((CH,), jnp.int32),
            pltpu.VMEM((CH, H), jnp.float32),
            pltpu.VMEM((CH, H), jnp.float32),
            pltpu.SemaphoreType.DMA,
            pltpu.SemaphoreType.DMA,
            pltpu.SemaphoreType.DMA,
            pltpu.SemaphoreType.DMA,
        ],
    )(_edge_scatter_body)


def _edge_scatter_body(src_hbm, dst_hbm, wtab_hbm, zeros_hbm, out_hbm,
                       acc_sh, srcb0, srcb1, dstb0, dstb1, rows0, rows1,
                       sem_g0, sem_g1, sem_i0, sem_i1):
    c = lax.axis_index("c")
    s = lax.axis_index("s")

    @pl.when(s < NZT)
    def _zero():
        pltpu.sync_copy(zeros_hbm, acc_sh.at[pl.ds(s * RPT, RPT)])

    @pl.when(s == NZT)
    def _zero_pad():
        pltpu.sync_copy(zeros_hbm.at[pl.ds(0, PADR)], acc_sh.at[pl.ds(N, PADR)])

    def idx_load(g, srcb, dstb, sem):
        pltpu.async_copy(src_hbm.at[s, g], srcb, sem)
        pltpu.async_copy(dst_hbm.at[s, g], dstb, sem)

    def idx_wait(g, srcb, dstb, sem):
        pltpu.make_async_copy(src_hbm.at[s, g], srcb, sem).wait()
        pltpu.make_async_copy(dst_hbm.at[s, g], dstb, sem).wait()

    def adjust(srcb):
        for j in range(CH // L):
            sl = pl.ds(j * L, L)
            srcb[sl] = srcb[sl] * 2 + c

    def gather_start(srcb, rows, sem):
        pltpu.async_copy(wtab_hbm.at[srcb], rows, sem)

    def gather_wait(srcb, rows, sem):
        pltpu.make_async_copy(wtab_hbm.at[srcb], rows, sem).wait()

    def scat(dstb, rows):
        pltpu.sync_copy(rows, acc_sh.at[dstb], add=True)

    plsc.subcore_barrier()

    idx_load(0, srcb0, dstb0, sem_i0)
    idx_wait(0, srcb0, dstb0, sem_i0)
    adjust(srcb0)
    gather_start(srcb0, rows0, sem_g0)
    idx_load(1, srcb1, dstb1, sem_i1)

    def step(i, carry):
        a = 2 * i
        idx_wait(a + 1, srcb1, dstb1, sem_i1)
        adjust(srcb1)
        gather_start(srcb1, rows1, sem_g1)
        gather_wait(srcb0, rows0, sem_g0)
        scat(dstb0, rows0)

        @pl.when(a + 2 < NCHUNK)
        def _even_next():
            idx_load(a + 2, srcb0, dstb0, sem_i0)
            idx_wait(a + 2, srcb0, dstb0, sem_i0)
            adjust(srcb0)
            gather_start(srcb0, rows0, sem_g0)

        gather_wait(srcb1, rows1, sem_g1)
        scat(dstb1, rows1)

        @pl.when(a + 3 < NCHUNK)
        def _odd_next():
            idx_load(a + 3, srcb1, dstb1, sem_i1)

        return carry

    lax.fori_loop(0, NCHUNK // 2, step, 0)
    plsc.subcore_barrier()

    @pl.when(s < NZT)
    def _writeback():
        pltpu.sync_copy(acc_sh.at[pl.ds(s * RPT, RPT)],
                        out_hbm.at[c, pl.ds(s * RPT, RPT)])


BLK = 400
_DIMS = (((1,), (1,)), ((), ()))


def _matT(t, W):
    return lax.dot_general(t, W, _DIMS, preferred_element_type=jnp.float32)


def _dcoefs(W, w):
    wm = W * w
    return lax.rsqrt(jnp.sum(wm * wm, axis=1) + 1e-8).reshape(1, D)


def _style(t, W, b, nz, w):
    a = _matT(t, W) + b
    a = _matT(a * w, W)
    a = a * _dcoefs(W, w) + nz + b
    return jnp.where(a >= 0, a, 0.01 * a)


def _dense_body(xl_ref, xr_ref, x_ref, w_ref, be_ref, W1_ref, b1_ref,
                W2_ref, b2_ref, Wn_ref, bn_ref, n0_ref, Wf1_ref, bf1_ref,
                n1_ref, Wf2_ref, bf2_ref, n2_ref, o_ref):
    w = w_ref[...]
    xn = _style(x_ref[...], Wn_ref[...], bn_ref[...], n0_ref[...], w)
    o = jnp.concatenate([xl_ref[...], xr_ref[...]], axis=1) + be_ref[...]
    o = (o + _matT(o, W1_ref[...]) + b1_ref[...]
         + xn + _matT(xn, W2_ref[...]) + b2_ref[...])
    o = jnp.maximum(o, 0.0)
    o = _style(o, Wf1_ref[...], bf1_ref[...], n1_ref[...], w)
    o = _style(o, Wf2_ref[...], bf2_ref[...], n2_ref[...], w)
    o_ref[...] = o


def _half_spec():
    return pl.BlockSpec((BLK, H), lambda i: (i, 0))


def _full_spec():
    return pl.BlockSpec((BLK, D), lambda i: (i, 0))


def _mat_spec():
    return pl.BlockSpec((D, D), lambda i: (0, 0))


def _vec_spec():
    return pl.BlockSpec((1, D), lambda i: (0, 0))


_dense = pl.pallas_call(
    _dense_body,
    grid=(N // BLK,),
    in_specs=[
        _half_spec(), _half_spec(), _full_spec(), _vec_spec(), _vec_spec(),
        _mat_spec(), _vec_spec(), _mat_spec(), _vec_spec(),
        _mat_spec(), _vec_spec(), _vec_spec(),
        _mat_spec(), _vec_spec(), _vec_spec(),
        _mat_spec(), _vec_spec(), _vec_spec(),
    ],
    out_specs=_full_spec(),
    out_shape=jax.ShapeDtypeStruct((N, D), jnp.float32),
)


def kernel(x, edge_index, w, W_edge, b_edge, W_cat1, b_cat1, W_cat2, b_cat2,
           W_node, b_node, ns_node, W_f1, b_f1, ns_f1, W_f2, b_f2, ns_f2):
    npad = EPAD - E
    src = jnp.concatenate([edge_index[0], jnp.zeros((npad,), jnp.int32)])
    dst = jnp.concatenate([edge_index[1], jnp.full((npad,), N, jnp.int32)])
    src = src.reshape(NS, NCHUNK, CH)
    dst = dst.reshape(NS, NCHUNK, CH)
    wtab = W_edge.reshape(2 * N, H)
    zeros = jnp.zeros((RPT, H), jnp.float32)

    nk = jax.random.key(7)
    n0 = jax.random.normal(jax.random.fold_in(nk, 0), (1, D), jnp.float32) * ns_node
    n1 = jax.random.normal(jax.random.fold_in(nk, 1), (1, D), jnp.float32) * ns_f1
    n2 = jax.random.normal(jax.random.fold_in(nk, 2), (1, D), jnp.float32) * ns_f2

    row = lambda v: v.reshape(1, D)
    halves = _make_edge_scatter()(src, dst, wtab, zeros)
    return _dense(halves[0], halves[1], x, row(w), row(b_edge),
                  W_cat1, row(b_cat1), W_cat2, row(b_cat2),
                  W_node, row(b_node), n0,
                  W_f1, row(b_f1), n1,
                  W_f2, row(b_f2), n2)


# speedup vs baseline: 1.0534x; 1.0534x over previous
import functools

import jax
import jax.numpy as jnp
from jax import lax
from jax.experimental import pallas as pl
from jax.experimental.pallas import tpu as pltpu
from jax.experimental.pallas import tpu_sc as plsc

N = 10000
D = 256
E = 160000
H = D // 2
NC = 2
NS = 16
L = 16
CH = 112
NCHUNK = 90
EPT = CH * NCHUNK
EPAD = NS * EPT
PADR = 16
NZT = 10
RPT = N // NZT


@functools.cache
def _make_edge_scatter():
    return functools.partial(
        pl.kernel,
        out_type=jax.ShapeDtypeStruct((NC, N, H), jnp.float32),
        mesh=plsc.VectorSubcoreMesh(
            core_axis_name="c", subcore_axis_name="s",
            num_cores=NC, num_subcores=NS,
        ),
        scratch_types=[
            pltpu.VMEM_SHARED((N + PADR, H), jnp.float32),
            pltpu.VMEM((CH,), jnp.int32),
            pltpu.VMEM((CH,), jnp.int32),
            pltpu.VMEM((CH,), jnp.int32),
            pltpu.VMEM((CH,), jnp.int32),
            pltpu.VMEM((CH, H), jnp.float32),
            pltpu.VMEM((CH, H), jnp.float32),
            pltpu.SemaphoreType.DMA,
            pltpu.SemaphoreType.DMA,
            pltpu.SemaphoreType.DMA,
            pltpu.SemaphoreType.DMA,
        ],
    )(_edge_scatter_body)


def _edge_scatter_body(src_hbm, dst_hbm, wtab_hbm, zeros_hbm, out_hbm,
                       acc_sh, srcb0, srcb1, dstb0, dstb1, rows0, rows1,
                       sem_g0, sem_g1, sem_i0, sem_i1):
    c = lax.axis_index("c")
    s = lax.axis_index("s")

    @pl.when(s < NZT)
    def _zero():
        pltpu.sync_copy(zeros_hbm, acc_sh.at[pl.ds(s * RPT, RPT)])

    @pl.when(s == NZT)
    def _zero_pad():
        pltpu.sync_copy(zeros_hbm.at[pl.ds(0, PADR)], acc_sh.at[pl.ds(N, PADR)])

    def idx_load(g, srcb, dstb, sem):
        pltpu.async_copy(src_hbm.at[s, g], srcb, sem)
        pltpu.async_copy(dst_hbm.at[s, g], dstb, sem)

    def idx_wait(g, srcb, dstb, sem):
        pltpu.make_async_copy(src_hbm.at[s, g], srcb, sem).wait()
        pltpu.make_async_copy(dst_hbm.at[s, g], dstb, sem).wait()

    def adjust(srcb):
        for j in range(CH // L):
            sl = pl.ds(j * L, L)
            srcb[sl] = srcb[sl] * 2 + c

    def gather_start(srcb, rows, sem):
        pltpu.async_copy(wtab_hbm.at[srcb], rows, sem)

    def gather_wait(srcb, rows, sem):
        pltpu.make_async_copy(wtab_hbm.at[srcb], rows, sem).wait()

    def scat(dstb, rows):
        pltpu.sync_copy(rows, acc_sh.at[dstb], add=True)

    plsc.subcore_barrier()

    idx_load(0, srcb0, dstb0, sem_i0)
    idx_wait(0, srcb0, dstb0, sem_i0)
    adjust(srcb0)
    gather_start(srcb0, rows0, sem_g0)
    idx_load(1, srcb1, dstb1, sem_i1)

    def step(i, carry):
        a = 2 * i
        idx_wait(a + 1, srcb1, dstb1, sem_i1)
        adjust(srcb1)
        gather_start(srcb1, rows1, sem_g1)
        gather_wait(srcb0, rows0, sem_g0)
        scat(dstb0, rows0)

        @pl.when(a + 2 < NCHUNK)
        def _even_next():
            idx_load(a + 2, srcb0, dstb0, sem_i0)
            idx_wait(a + 2, srcb0, dstb0, sem_i0)
            adjust(srcb0)
            gather_start(srcb0, rows0, sem_g0)

        gather_wait(srcb1, rows1, sem_g1)
        scat(dstb1, rows1)

        @pl.when(a + 3 < NCHUNK)
        def _odd_next():
            idx_load(a + 3, srcb1, dstb1, sem_i1)

        return carry

    lax.fori_loop(0, NCHUNK // 2, step, 0)
    plsc.subcore_barrier()

    @pl.when(s < NZT)
    def _writeback():
        pltpu.sync_copy(acc_sh.at[pl.ds(s * RPT, RPT)],
                        out_hbm.at[c, pl.ds(s * RPT, RPT)])


BLK = 1000
_DIMS = (((1,), (1,)), ((), ()))


def _matT(t, W):
    return lax.dot_general(t, W, _DIMS, preferred_element_type=jnp.float32)


def _dcoefs(W, w):
    wm = W * w
    return lax.rsqrt(jnp.sum(wm * wm, axis=1) + 1e-8).reshape(1, D)


def _style(t, W, b, nz, w):
    a = _matT(t, W) + b
    a = _matT(a * w, W)
    a = a * _dcoefs(W, w) + nz + b
    return jnp.where(a >= 0, a, 0.01 * a)


def _dense_pre_body(x_ref, w_ref, Wn_ref, bn_ref, n0_ref, W2_ref, b2_ref,
                    s_ref):
    w = w_ref[...]
    xn = _style(x_ref[...], Wn_ref[...], bn_ref[...], n0_ref[...], w)
    s_ref[...] = xn + _matT(xn, W2_ref[...]) + b2_ref[...]


def _dense_post_body(xl_ref, xr_ref, s_ref, w_ref, be_ref, W1_ref, b1_ref,
                     Wf1_ref, bf1_ref, n1_ref, Wf2_ref, bf2_ref, n2_ref,
                     o_ref):
    w = w_ref[...]
    o = jnp.concatenate([xl_ref[...], xr_ref[...]], axis=1) + be_ref[...]
    o = o + _matT(o, W1_ref[...]) + b1_ref[...] + s_ref[...]
    o = jnp.maximum(o, 0.0)
    o = _style(o, Wf1_ref[...], bf1_ref[...], n1_ref[...], w)
    o = _style(o, Wf2_ref[...], bf2_ref[...], n2_ref[...], w)
    o_ref[...] = o


def _half_spec():
    return pl.BlockSpec((BLK, H), lambda i: (i, 0))


def _full_spec():
    return pl.BlockSpec((BLK, D), lambda i: (i, 0))


def _mat_spec():
    return pl.BlockSpec((D, D), lambda i: (0, 0))


def _vec_spec():
    return pl.BlockSpec((1, D), lambda i: (0, 0))


_dense_pre = pl.pallas_call(
    _dense_pre_body,
    grid=(N // BLK,),
    in_specs=[
        _full_spec(), _vec_spec(),
        _mat_spec(), _vec_spec(), _vec_spec(),
        _mat_spec(), _vec_spec(),
    ],
    out_specs=_full_spec(),
    out_shape=jax.ShapeDtypeStruct((N, D), jnp.float32),
)

_dense_post = pl.pallas_call(
    _dense_post_body,
    grid=(N // BLK,),
    in_specs=[
        _half_spec(), _half_spec(), _full_spec(), _vec_spec(), _vec_spec(),
        _mat_spec(), _vec_spec(),
        _mat_spec(), _vec_spec(), _vec_spec(),
        _mat_spec(), _vec_spec(), _vec_spec(),
    ],
    out_specs=_full_spec(),
    out_shape=jax.ShapeDtypeStruct((N, D), jnp.float32),
)


def kernel(x, edge_index, w, W_edge, b_edge, W_cat1, b_cat1, W_cat2, b_cat2,
           W_node, b_node, ns_node, W_f1, b_f1, ns_f1, W_f2, b_f2, ns_f2):
    npad = EPAD - E
    src = jnp.concatenate([edge_index[0], jnp.zeros((npad,), jnp.int32)])
    dst = jnp.concatenate([edge_index[1], jnp.full((npad,), N, jnp.int32)])
    src = src.reshape(NS, NCHUNK, CH)
    dst = dst.reshape(NS, NCHUNK, CH)
    wtab = W_edge.reshape(2 * N, H)
    zeros = jnp.zeros((RPT, H), jnp.float32)

    nk = jax.random.key(7)
    n0 = jax.random.normal(jax.random.fold_in(nk, 0), (1, D), jnp.float32) * ns_node
    n1 = jax.random.normal(jax.random.fold_in(nk, 1), (1, D), jnp.float32) * ns_f1
    n2 = jax.random.normal(jax.random.fold_in(nk, 2), (1, D), jnp.float32) * ns_f2

    row = lambda v: v.reshape(1, D)
    s = _dense_pre(x, row(w), W_node, row(b_node), n0, W_cat2, row(b_cat2))
    halves = _make_edge_scatter()(src, dst, wtab, zeros)
    return _dense_post(halves[0], halves[1], s, row(w), row(b_edge),
                       W_cat1, row(b_cat1),
                       W_f1, row(b_f1), n1,
                       W_f2, row(b_f2), n2)


# speedup vs baseline: 1.0623x; 1.0084x over previous
import functools

import jax
import jax.numpy as jnp
from jax import lax
from jax.experimental import pallas as pl
from jax.experimental.pallas import tpu as pltpu
from jax.experimental.pallas import tpu_sc as plsc

N = 10000
D = 256
E = 160000
H = D // 2
NC = 2
NS = 16
L = 16
CH = 112
NCHUNK = 90
EPT = CH * NCHUNK
EPAD = NS * EPT
PADR = 16
NZT = 10
RPT = N // NZT


@functools.cache
def _make_edge_scatter():
    return functools.partial(
        pl.kernel,
        out_type=jax.ShapeDtypeStruct((NC, N, H), jnp.float32),
        mesh=plsc.VectorSubcoreMesh(
            core_axis_name="c", subcore_axis_name="s",
            num_cores=NC, num_subcores=NS,
        ),
        scratch_types=[
            pltpu.VMEM_SHARED((N + PADR, H), jnp.float32),
            pltpu.VMEM((CH,), jnp.int32),
            pltpu.VMEM((CH,), jnp.int32),
            pltpu.VMEM((CH,), jnp.int32),
            pltpu.VMEM((CH,), jnp.int32),
            pltpu.VMEM((CH, H), jnp.float32),
            pltpu.VMEM((CH, H), jnp.float32),
            pltpu.SemaphoreType.DMA,
            pltpu.SemaphoreType.DMA,
            pltpu.SemaphoreType.DMA,
            pltpu.SemaphoreType.DMA,
        ],
    )(_edge_scatter_body)


def _edge_scatter_body(src_hbm, dst_hbm, wtab_hbm, zeros_hbm, out_hbm,
                       acc_sh, srcb0, srcb1, dstb0, dstb1, rows0, rows1,
                       sem_g0, sem_g1, sem_i0, sem_i1):
    c = lax.axis_index("c")
    s = lax.axis_index("s")

    @pl.when(s < NZT)
    def _zero():
        pltpu.sync_copy(zeros_hbm, acc_sh.at[pl.ds(s * RPT, RPT)])

    @pl.when(s == NZT)
    def _zero_pad():
        pltpu.sync_copy(zeros_hbm.at[pl.ds(0, PADR)], acc_sh.at[pl.ds(N, PADR)])

    def idx_load(g, srcb, dstb, sem):
        pltpu.async_copy(src_hbm.at[s, g], srcb, sem)
        pltpu.async_copy(dst_hbm.at[s, g], dstb, sem)

    def idx_wait(g, srcb, dstb, sem):
        pltpu.make_async_copy(src_hbm.at[s, g], srcb, sem).wait()
        pltpu.make_async_copy(dst_hbm.at[s, g], dstb, sem).wait()

    def adjust(srcb):
        for j in range(CH // L):
            sl = pl.ds(j * L, L)
            srcb[sl] = srcb[sl] * 2 + c

    def gather_start(srcb, rows, sem):
        pltpu.async_copy(wtab_hbm.at[srcb], rows, sem)

    def gather_wait(srcb, rows, sem):
        pltpu.make_async_copy(wtab_hbm.at[srcb], rows, sem).wait()

    def scat(dstb, rows):
        pltpu.sync_copy(rows, acc_sh.at[dstb], add=True)

    plsc.subcore_barrier()

    idx_load(0, srcb0, dstb0, sem_i0)
    idx_wait(0, srcb0, dstb0, sem_i0)
    adjust(srcb0)
    gather_start(srcb0, rows0, sem_g0)
    idx_load(1, srcb1, dstb1, sem_i1)

    def step(i, carry):
        a = 2 * i
        idx_wait(a + 1, srcb1, dstb1, sem_i1)
        adjust(srcb1)
        gather_start(srcb1, rows1, sem_g1)
        gather_wait(srcb0, rows0, sem_g0)
        scat(dstb0, rows0)

        @pl.when(a + 2 < NCHUNK)
        def _even_next():
            idx_load(a + 2, srcb0, dstb0, sem_i0)
            idx_wait(a + 2, srcb0, dstb0, sem_i0)
            adjust(srcb0)
            gather_start(srcb0, rows0, sem_g0)

        gather_wait(srcb1, rows1, sem_g1)
        scat(dstb1, rows1)

        @pl.when(a + 3 < NCHUNK)
        def _odd_next():
            idx_load(a + 3, srcb1, dstb1, sem_i1)

        return carry

    lax.fori_loop(0, NCHUNK // 2, step, 0)
    plsc.subcore_barrier()

    @pl.when(s < NZT)
    def _writeback():
        pltpu.sync_copy(acc_sh.at[pl.ds(s * RPT, RPT)],
                        out_hbm.at[c, pl.ds(s * RPT, RPT)])


BLK = 2000
_DIMS = (((1,), (1,)), ((), ()))


def _matT(t, W):
    return lax.dot_general(t, W, _DIMS, preferred_element_type=jnp.float32)


def _dcoefs(W, w):
    wm = W * w
    return lax.rsqrt(jnp.sum(wm * wm, axis=1) + 1e-8).reshape(1, D)


def _style(t, W, b, nz, w):
    a = _matT(t, W) + b
    a = _matT(a * w, W)
    a = a * _dcoefs(W, w) + nz + b
    return jnp.where(a >= 0, a, 0.01 * a)


def _dense_pre_body(x_ref, w_ref, Wn_ref, bn_ref, n0_ref, W2_ref, b2_ref,
                    s_ref):
    w = w_ref[...]
    xn = _style(x_ref[...], Wn_ref[...], bn_ref[...], n0_ref[...], w)
    s_ref[...] = xn + _matT(xn, W2_ref[...]) + b2_ref[...]


def _dense_post_body(xl_ref, xr_ref, s_ref, w_ref, be_ref, W1_ref, b1_ref,
                     Wf1_ref, bf1_ref, n1_ref, Wf2_ref, bf2_ref, n2_ref,
                     o_ref):
    w = w_ref[...]
    o = jnp.concatenate([xl_ref[...], xr_ref[...]], axis=1) + be_ref[...]
    o = o + _matT(o, W1_ref[...]) + b1_ref[...] + s_ref[...]
    o = jnp.maximum(o, 0.0)
    o = _style(o, Wf1_ref[...], bf1_ref[...], n1_ref[...], w)
    o = _style(o, Wf2_ref[...], bf2_ref[...], n2_ref[...], w)
    o_ref[...] = o


def _half_spec():
    return pl.BlockSpec((BLK, H), lambda i: (i, 0))


def _full_spec():
    return pl.BlockSpec((BLK, D), lambda i: (i, 0))


def _mat_spec():
    return pl.BlockSpec((D, D), lambda i: (0, 0))


def _vec_spec():
    return pl.BlockSpec((1, D), lambda i: (0, 0))


_dense_pre = pl.pallas_call(
    _dense_pre_body,
    grid=(N // BLK,),
    in_specs=[
        _full_spec(), _vec_spec(),
        _mat_spec(), _vec_spec(), _vec_spec(),
        _mat_spec(), _vec_spec(),
    ],
    out_specs=_full_spec(),
    out_shape=jax.ShapeDtypeStruct((N, D), jnp.float32),
)

_dense_post = pl.pallas_call(
    _dense_post_body,
    grid=(N // BLK,),
    in_specs=[
        _half_spec(), _half_spec(), _full_spec(), _vec_spec(), _vec_spec(),
        _mat_spec(), _vec_spec(),
        _mat_spec(), _vec_spec(), _vec_spec(),
        _mat_spec(), _vec_spec(), _vec_spec(),
    ],
    out_specs=_full_spec(),
    out_shape=jax.ShapeDtypeStruct((N, D), jnp.float32),
)


def kernel(x, edge_index, w, W_edge, b_edge, W_cat1, b_cat1, W_cat2, b_cat2,
           W_node, b_node, ns_node, W_f1, b_f1, ns_f1, W_f2, b_f2, ns_f2):
    npad = EPAD - E
    src = jnp.concatenate([edge_index[0], jnp.zeros((npad,), jnp.int32)])
    dst = jnp.concatenate([edge_index[1], jnp.full((npad,), N, jnp.int32)])
    src = src.reshape(NS, NCHUNK, CH)
    dst = dst.reshape(NS, NCHUNK, CH)
    wtab = W_edge.reshape(2 * N, H)
    zeros = jnp.zeros((RPT, H), jnp.float32)

    nk = jax.random.key(7)
    n0 = jax.random.normal(jax.random.fold_in(nk, 0), (1, D), jnp.float32) * ns_node
    n1 = jax.random.normal(jax.random.fold_in(nk, 1), (1, D), jnp.float32) * ns_f1
    n2 = jax.random.normal(jax.random.fold_in(nk, 2), (1, D), jnp.float32) * ns_f2

    row = lambda v: v.reshape(1, D)
    s = _dense_pre(x, row(w), W_node, row(b_node), n0, W_cat2, row(b_cat2))
    halves = _make_edge_scatter()(src, dst, wtab, zeros)
    return _dense_post(halves[0], halves[1], s, row(w), row(b_edge),
                       W_cat1, row(b_cat1),
                       W_f1, row(b_f1), n1,
                       W_f2, row(b_f2), n2)
